# Initial kernel scaffold; baseline (speedup 1.0000x reference)
import jax
import jax.numpy as jnp
from jax.experimental import pallas as pl


def kernel(z_e_x, W):
    raise NotImplementedError("write your pallas kernel here")


# speedup vs baseline: 1.2600x; 1.2600x over previous
import jax
import jax.numpy as jnp
from jax import lax
from jax.experimental import pallas as pl

_K = 8192
_D = 256
_TOK_BLK = 512


def _vq_body(zn_ref, z_ref, w_ref, wn_ref, out_ref):
    z = z_ref[...]
    w = w_ref[...]
    dot = lax.dot_general(
        z, w, (((1,), (1,)), ((), ())),
        preferred_element_type=jnp.float32,
    )
    dist = (zn_ref[...] - 2.0 * dot) + wn_ref[...]
    idx = jnp.argmin(dist, axis=1).astype(jnp.int32)
    out_ref[0, 0, :] = idx


def kernel(z_e_x, W):
    B, C, H, Wd = z_e_x.shape
    n_tok = B * H * Wd
    flat = jnp.transpose(z_e_x, (0, 2, 3, 1)).reshape(n_tok, C)
    znorm = jnp.sum(flat * flat, axis=1, keepdims=True)
    wnorm = jnp.sum(W * W, axis=1)[None, :]
    flat_bf = flat.astype(jnp.bfloat16)
    W_bf = W.astype(jnp.bfloat16)

    n_blk = n_tok // _TOK_BLK
    idx = pl.pallas_call(
        _vq_body,
        grid=(n_blk,),
        in_specs=[
            pl.BlockSpec((_TOK_BLK, 1), lambda i: (i, 0)),
            pl.BlockSpec((_TOK_BLK, _D), lambda i: (i, 0)),
            pl.BlockSpec((_K, _D), lambda i: (0, 0)),
            pl.BlockSpec((1, _K), lambda i: (0, 0)),
        ],
        out_specs=pl.BlockSpec((1, 1, _TOK_BLK), lambda i: (i, 0, 0)),
        out_shape=jax.ShapeDtypeStruct((n_blk, 1, _TOK_BLK), jnp.int32),
    )(znorm, flat_bf, W_bf, wnorm)
    return idx.reshape(B, H, Wd)


# speedup vs baseline: 1.4006x; 1.1116x over previous
import jax
import jax.numpy as jnp
from jax import lax
from jax.experimental import pallas as pl

_K = 8192
_D = 256
_TOK_BLK = 512


def _vq_body(zn_ref, z_ref, w_ref, wn_ref, out_ref):
    z = z_ref[...]
    w = w_ref[...]
    dot2 = lax.dot_general(
        z, w, (((1,), (1,)), ((), ())),
        preferred_element_type=jnp.float32,
    )
    dist = (zn_ref[...] - dot2) + wn_ref[...]
    idx = jnp.argmin(dist, axis=1).astype(jnp.int32)
    out_ref[0, 0, :] = idx


def kernel(z_e_x, W):
    B, C, H, Wd = z_e_x.shape
    n_tok = B * H * Wd
    flat = jnp.transpose(z_e_x, (0, 2, 3, 1)).reshape(n_tok, C)
    znorm = jnp.sum(flat * flat, axis=1, keepdims=True)
    wnorm = jnp.sum(W * W, axis=1)[None, :]
    flat_bf = flat.astype(jnp.bfloat16)
    W_bf = (2.0 * W).astype(jnp.bfloat16)

    n_blk = n_tok // _TOK_BLK
    idx = pl.pallas_call(
        _vq_body,
        grid=(n_blk,),
        in_specs=[
            pl.BlockSpec((_TOK_BLK, 1), lambda i: (i, 0)),
            pl.BlockSpec((_TOK_BLK, _D), lambda i: (i, 0)),
            pl.BlockSpec((_K, _D), lambda i: (0, 0)),
            pl.BlockSpec((1, _K), lambda i: (0, 0)),
        ],
        out_specs=pl.BlockSpec((1, 1, _TOK_BLK), lambda i: (i, 0, 0)),
        out_shape=jax.ShapeDtypeStruct((n_blk, 1, _TOK_BLK), jnp.int32),
    )(znorm, flat_bf, W_bf, wnorm)
    return idx.reshape(B, H, Wd)


# speedup vs baseline: 1.4043x; 1.0027x over previous
import jax
import jax.numpy as jnp
from jax import lax
from jax.experimental import pallas as pl
from jax.experimental.pallas import tpu as pltpu

_K = 8192
_D = 256
_TOK_BLK = 512


def _vq_body(zn_ref, z_ref, w_ref, wn_ref, out_ref):
    z = z_ref[...]
    w = w_ref[...]
    dot2 = lax.dot_general(
        z, w, (((1,), (1,)), ((), ())),
        preferred_element_type=jnp.float32,
    )
    dist = (zn_ref[...] - dot2) + wn_ref[...]
    idx = jnp.argmin(dist, axis=1).astype(jnp.int32)
    out_ref[0, 0, :] = idx


def kernel(z_e_x, W):
    B, C, H, Wd = z_e_x.shape
    n_tok = B * H * Wd
    flat = jnp.transpose(z_e_x, (0, 2, 3, 1)).reshape(n_tok, C)
    znorm = jnp.sum(flat * flat, axis=1, keepdims=True)
    wnorm = jnp.sum(W * W, axis=1)[None, :]
    flat_bf = flat.astype(jnp.bfloat16)
    W_bf = (2.0 * W).astype(jnp.bfloat16)

    n_blk = n_tok // _TOK_BLK
    idx = pl.pallas_call(
        _vq_body,
        grid=(n_blk,),
        in_specs=[
            pl.BlockSpec((_TOK_BLK, 1), lambda i: (i, 0)),
            pl.BlockSpec((_TOK_BLK, _D), lambda i: (i, 0)),
            pl.BlockSpec((_K, _D), lambda i: (0, 0)),
            pl.BlockSpec((1, _K), lambda i: (0, 0)),
        ],
        out_specs=pl.BlockSpec((1, 1, _TOK_BLK), lambda i: (i, 0, 0)),
        out_shape=jax.ShapeDtypeStruct((n_blk, 1, _TOK_BLK), jnp.int32),
        compiler_params=pltpu.CompilerParams(
            dimension_semantics=("parallel",)),
    )(znorm, flat_bf, W_bf, wnorm)
    return idx.reshape(B, H, Wd)


# speedup vs baseline: 1.4355x; 1.0222x over previous
import jax
import jax.numpy as jnp
from jax import lax
from jax.experimental import pallas as pl
from jax.experimental.pallas import tpu as pltpu

_K = 8192
_D = 256
_TOK_BLK = 1024


def _vq_body(zn_ref, z_ref, w_ref, wn_ref, out_ref):
    z = z_ref[...]
    w = w_ref[...]
    dot2 = lax.dot_general(
        z, w, (((1,), (1,)), ((), ())),
        preferred_element_type=jnp.float32,
    )
    dist = (zn_ref[...] - dot2) + wn_ref[...]
    idx = jnp.argmin(dist, axis=1).astype(jnp.int32)
    out_ref[0, 0, :] = idx


def kernel(z_e_x, W):
    B, C, H, Wd = z_e_x.shape
    n_tok = B * H * Wd
    flat = jnp.transpose(z_e_x, (0, 2, 3, 1)).reshape(n_tok, C)
    znorm = jnp.sum(flat * flat, axis=1, keepdims=True)
    wnorm = jnp.sum(W * W, axis=1)[None, :]
    flat_bf = flat.astype(jnp.bfloat16)
    W_bf = (2.0 * W).astype(jnp.bfloat16)

    n_blk = n_tok // _TOK_BLK
    idx = pl.pallas_call(
        _vq_body,
        grid=(n_blk,),
        in_specs=[
            pl.BlockSpec((_TOK_BLK, 1), lambda i: (i, 0)),
            pl.BlockSpec((_TOK_BLK, _D), lambda i: (i, 0)),
            pl.BlockSpec((_K, _D), lambda i: (0, 0)),
            pl.BlockSpec((1, _K), lambda i: (0, 0)),
        ],
        out_specs=pl.BlockSpec((1, 1, _TOK_BLK), lambda i: (i, 0, 0)),
        out_shape=jax.ShapeDtypeStruct((n_blk, 1, _TOK_BLK), jnp.int32),
        compiler_params=pltpu.CompilerParams(
            dimension_semantics=("parallel",)),
    )(znorm, flat_bf, W_bf, wnorm)
    return idx.reshape(B, H, Wd)
